# Initial kernel scaffold; baseline (speedup 1.0000x reference)
import jax
import jax.numpy as jnp
from jax.experimental import pallas as pl


def kernel(input, adj, W, hop_logits):
    raise NotImplementedError("write your pallas kernel here")


# speedup vs baseline: 4.8621x; 4.8621x over previous
import jax
import jax.numpy as jnp
from jax.experimental import pallas as pl
from jax.experimental.pallas import tpu as pltpu

N = 4096
D = 128
RB = 512


def _matmul_kernel(x_ref, w_ref, o_ref):
    o_ref[...] = jnp.dot(x_ref[...], w_ref[...],
                         preferred_element_type=jnp.float32)


def _hop1_kernel(a_ref, s_ref, sblk_ref, hw_ref, o_ref):
    hw0 = hw_ref[0, 0]
    hw1 = hw_ref[0, 1]
    ms = jnp.dot(a_ref[...], s_ref[...], preferred_element_type=jnp.float32)
    o_ref[...] = hw0 * sblk_ref[...] + hw1 * ms


def _hop2_kernel(a_ref, u_ref, x_ref, o_ref):
    y = jnp.dot(a_ref[...], u_ref[...], preferred_element_type=jnp.float32)
    o_ref[...] = jnp.maximum(y + x_ref[...], 0.0)


def kernel(input, adj, W, hop_logits):
    hw = jax.nn.softmax(hop_logits, axis=0)
    hw2 = hw.reshape(1, 2)

    support = pl.pallas_call(
        _matmul_kernel,
        out_shape=jax.ShapeDtypeStruct((N, D), jnp.float32),
    )(input, W)

    grid = N // RB
    u = pl.pallas_call(
        _hop1_kernel,
        grid=(grid,),
        in_specs=[
            pl.BlockSpec((RB, N), lambda i: (i, 0)),
            pl.BlockSpec((N, D), lambda i: (0, 0)),
            pl.BlockSpec((RB, D), lambda i: (i, 0)),
            pl.BlockSpec((1, 2), lambda i: (0, 0)),
        ],
        out_specs=pl.BlockSpec((RB, D), lambda i: (i, 0)),
        out_shape=jax.ShapeDtypeStruct((N, D), jnp.float32),
    )(adj, support, support, hw2)

    out = pl.pallas_call(
        _hop2_kernel,
        grid=(grid,),
        in_specs=[
            pl.BlockSpec((RB, N), lambda i: (i, 0)),
            pl.BlockSpec((N, D), lambda i: (0, 0)),
            pl.BlockSpec((RB, D), lambda i: (i, 0)),
        ],
        out_specs=pl.BlockSpec((RB, D), lambda i: (i, 0)),
        out_shape=jax.ShapeDtypeStruct((N, D), jnp.float32),
    )(adj, u, input)

    return out
